# Initial kernel scaffold; baseline (speedup 1.0000x reference)
import jax
import jax.numpy as jnp
from jax.experimental import pallas as pl


def kernel(mem, obs_feat, new_feat, obs_slots, new_slots, active_ids, active_det_idx):
    raise NotImplementedError("write your pallas kernel here")


# speedup vs baseline: 20.1856x; 20.1856x over previous
import functools

import jax
import jax.numpy as jnp
from jax import lax
from jax.experimental import pallas as pl
from jax.experimental.pallas import tpu as pltpu
from jax.experimental.pallas import tpu_sc as plsc

M = 524288
D = 128
A = 131072
N = 65536
TOT = A + N

NC = 2
NS = 16
NW = NC * NS
L = 16

RPW = M // NW
ICH = 8192
CCH = 256


def _fori(lo, hi, body, init):
    return lax.fori_loop(jnp.int32(lo), jnp.int32(hi), body, init)


def _sc_scatter_body(t_hbm, slots_hbm, out_hbm, winner, idxbuf, srcrow,
                     rowbuf, sem):
    wid = lax.axis_index("s") * NC + lax.axis_index("c")
    base = wid * RPW
    iota = lax.iota(jnp.int32, L)
    neg1 = jnp.full((L,), -1, jnp.int32)

    def init_body(k, carry):
        winner[pl.ds(k * L, L)] = neg1
        return carry

    _fori(0, RPW // L, init_body, 0)

    def idx_chunk(c, carry):
        pltpu.sync_copy(slots_hbm.at[pl.ds(c * ICH, ICH)], idxbuf)

        def vec_body(v, inner):
            s = idxbuf[pl.ds(v * L, L)]
            local = s - base
            m = (local >= 0) & (local < RPW)
            localc = local & (RPW - 1)
            ivec = c * ICH + v * L + iota

            def wcond(nleft):
                return nleft > 0

            def wbody(_):
                cur = plsc.load_gather(winner, [localc], mask=m)
                upd = m & (ivec > cur)
                plsc.store_scatter(winner, [localc], ivec, mask=upd)
                return jnp.sum(upd.astype(jnp.int32), dtype=jnp.int32)

            lax.while_loop(wcond, wbody, jnp.sum(m.astype(jnp.int32), dtype=jnp.int32))
            return inner

        _fori(0, ICH // L, vec_body, 0)
        return carry

    _fori(0, TOT // ICH, idx_chunk, 0)

    def out_chunk(c, carry):
        row0 = c * CCH

        def vec_body(v, inner):
            w = winner[pl.ds(row0 + v * L, L)]
            linear = base + row0 + v * L + iota
            srcrow[pl.ds(v * L, L)] = jnp.where(w >= 0, M + w, linear)
            return inner

        _fori(0, CCH // L, vec_body, 0)
        pltpu.async_copy(t_hbm.at[srcrow], rowbuf, sem).wait()
        pltpu.sync_copy(rowbuf, out_hbm.at[pl.ds(base + row0, CCH)])
        return carry

    _fori(0, RPW // CCH, out_chunk, 0)


_sc_scatter = functools.partial(
    pl.kernel,
    out_type=jax.ShapeDtypeStruct((M, D), jnp.float32),
    mesh=plsc.VectorSubcoreMesh(core_axis_name="c", subcore_axis_name="s"),
    scratch_types=[
        pltpu.VMEM((RPW,), jnp.int32),
        pltpu.VMEM((ICH,), jnp.int32),
        pltpu.VMEM((CCH,), jnp.int32),
        pltpu.VMEM((CCH, D), jnp.float32),
        pltpu.SemaphoreType.DMA,
    ],
    compiler_params=pltpu.CompilerParams(needs_layout_passes=False),
)(_sc_scatter_body)


def _ids_body(act_ref, out_ref):
    act = act_ref[...]
    mx = jnp.max(act)
    out_ref[0:A // D, :] = act
    r = lax.broadcasted_iota(jnp.int32, (N // D, D), 0)
    c = lax.broadcasted_iota(jnp.int32, (N // D, D), 1)
    out_ref[A // D:(A + N) // D, :] = mx + 1 + r * D + c


_ids_kernel = pl.pallas_call(
    _ids_body,
    out_shape=jax.ShapeDtypeStruct(((A + N) // D, D), jnp.int32),
)


def kernel(mem, obs_feat, new_feat, obs_slots, new_slots, active_ids,
           active_det_idx):
    slots = jnp.concatenate([obs_slots, new_slots]).astype(jnp.int32)
    table = jnp.concatenate([mem, obs_feat, new_feat], axis=0)
    new_mem = _sc_scatter(table, slots)
    act2d = active_ids.astype(jnp.int32).reshape(A // D, D)
    ids = _ids_kernel(act2d).reshape(-1).astype(active_ids.dtype)
    return (new_mem, ids)
